# baseline (speedup 1.0000x reference)
import jax
import jax.numpy as jnp
from jax.experimental import pallas as pl
from jax.experimental.pallas import tpu as pltpu
from jax.experimental.pallas import tpu_sc as plsc

_DECAY = 0.9
_OMD = 1.0 - _DECAY
_MATURITY = 500
_REINIT_DIV = 100
_KMAX = 64
_I32MAX = 2**31 - 1


def _sc_copy_kernel(w_hbm, o_hbm, *, rows_per_subcore):
    sid = jax.lax.axis_index("c") * 16 + jax.lax.axis_index("s")
    start = sid * rows_per_subcore
    pltpu.sync_copy(w_hbm.at[pl.ds(start, rows_per_subcore)],
                    o_hbm.at[pl.ds(start, rows_per_subcore)])


def _stream_kernel(w_ref, f_ref, rm_ref, outg_ref, inc_ref,
                   rmnew_ref, cas_ref, *, inv_b, nwb):
    s = pl.program_id(0)

    @pl.when(s < nwb)
    def _weight_phase():
        aw = jnp.abs(w_ref[...])
        outg_ref[...] = jnp.sum(aw, axis=0)[None, None, :]
        inc_ref[...] = jnp.sum(aw, axis=1, keepdims=True)

    @pl.when(s >= nwb)
    def _features_phase():
        f = f_ref[...]
        colsum = jnp.sum(f, axis=0, keepdims=True)
        rm_new = _DECAY * rm_ref[...] + _OMD * (colsum * inv_b)
        rmnew_ref[...] = rm_new
        cas_ref[...] = jnp.sum(jnp.abs(f - rm_new), axis=0, keepdims=True)


def _excl_prefix_sum_row(x):
    n = x.shape[1]
    s = x
    sh = 1
    while sh < n:
        shifted = jnp.concatenate(
            [jnp.zeros((1, sh), x.dtype), s[:, :n - sh]], axis=1)
        s = s + shifted
        sh *= 2
    return s - x


def _update_select_kernel(cas_ref, outg_ref, incr_ref, u_ref, ages_ref,
                          bias_ref, w_ref, unew_ref, bnew_ref, anew_ref,
                          wout_ref, zeros_ref, sems, *, inv_bin, out_n):
    i32 = jnp.int32
    outgoing = jnp.sum(outg_ref[...], axis=0)
    c = jnp.sum(outgoing * cas_ref[...]) * inv_bin
    u_new = _DECAY * u_ref[...] + _OMD * (c / (incr_ref[...] + 1e-8))
    unew_ref[...] = u_new

    ages = ages_ref[...]
    mature = ages > _MATURITY
    num_mature = jnp.sum(mature.astype(i32))
    num_reinit = num_mature // _REINIT_DIV
    r = jnp.maximum(num_reinit, 1)

    key = jnp.where(mature, jax.lax.bitcast_convert_type(u_new, i32),
                    i32(_I32MAX))

    def bisect(_, lohi):
        lo, hi = lohi
        mid = (lo + hi) // 2
        ge = jnp.sum((key <= mid).astype(i32)) >= r
        return jnp.where(ge, lo, mid), jnp.where(ge, mid, hi)

    _, v = jax.lax.fori_loop(0, 31, bisect, (i32(-1), i32(_I32MAX)))

    c_lt = jnp.sum((key < v).astype(i32))
    tie = key == v
    tie_rank = _excl_prefix_sum_row(tie.astype(i32))
    sel = ((key < v) | (tie & (tie_rank < (r - c_lt)))) & (num_reinit > 0)

    bnew_ref[...] = jnp.where(sel, 0.0, bias_ref[...])
    anew_ref[...] = jnp.where(sel, 0, ages) + 1

    slot = _excl_prefix_sum_row(sel.astype(i32))
    i_row = jax.lax.broadcasted_iota(i32, (1, out_n), 1)
    zeros_ref[...] = jnp.zeros_like(zeros_ref)
    idxs = []
    for k in range(_KMAX):
        hit = sel & (slot == k)
        idx_k = jnp.sum(jnp.where(hit, i_row + 1, 0)) - 1
        idxs.append(idx_k)

        @pl.when(idx_k >= 0)
        def _():
            pltpu.make_async_copy(
                zeros_ref, wout_ref.at[pl.ds(idx_k, 1), :], sems.at[k],
            ).start()
    for k in range(_KMAX):
        @pl.when(idxs[k] >= 0)
        def _():
            pltpu.make_async_copy(
                zeros_ref, wout_ref.at[pl.ds(idxs[k], 1), :], sems.at[k],
            ).wait()


def kernel(features, weight, bias, utilities, running_mean, ages):
    B, IN = features.shape
    OUT = weight.shape[0]
    CB = 512
    RB = 256
    NB = OUT // RB
    f32 = jnp.float32

    rm2 = running_mean.reshape(1, IN)
    NWB = NB
    NFB = IN // CB

    sc_kernel = pl.kernel(
        lambda w, o: _sc_copy_kernel(w, o, rows_per_subcore=OUT // 32),
        out_type=jax.ShapeDtypeStruct((OUT, IN), f32),
        mesh=plsc.VectorSubcoreMesh(core_axis_name="c", subcore_axis_name="s"),
    )
    w_copy = sc_kernel(weight)

    wlast = NWB - 1
    outg_part, incoming, rm_new2, cas = pl.pallas_call(
        lambda w, f, r, o1, o2, o4, o5: _stream_kernel(
            w, f, r, o1, o2, o4, o5, inv_b=1.0 / B, nwb=NWB),
        grid=(NWB + NFB,),
        in_specs=[
            pl.BlockSpec((RB, IN), lambda s: (jnp.minimum(s, wlast), 0)),
            pl.BlockSpec((B, CB), lambda s: (0, jnp.maximum(s - NWB, 0))),
            pl.BlockSpec((1, CB), lambda s: (0, jnp.maximum(s - NWB, 0))),
        ],
        out_specs=[
            pl.BlockSpec((1, 1, IN), lambda s: (jnp.minimum(s, wlast), 0, 0)),
            pl.BlockSpec((RB, 1), lambda s: (jnp.minimum(s, wlast), 0)),
            pl.BlockSpec((1, CB), lambda s: (0, jnp.maximum(s - NWB, 0))),
            pl.BlockSpec((1, CB), lambda s: (0, jnp.maximum(s - NWB, 0))),
        ],
        out_shape=[
            jax.ShapeDtypeStruct((NB, 1, IN), f32),
            jax.ShapeDtypeStruct((OUT, 1), f32),
            jax.ShapeDtypeStruct((1, IN), f32),
            jax.ShapeDtypeStruct((1, IN), f32),
        ],
    )(weight, features, rm2)

    vm = pl.BlockSpec(memory_space=pltpu.MemorySpace.VMEM)
    u_new, bias_new, ages_new, weight_new = pl.pallas_call(
        lambda ca, og, ic, u, ag, b, w, un, bn, an, wo, zr, sm:
            _update_select_kernel(
                ca, og, ic, u, ag, b, w, un, bn, an, wo, zr, sm,
                inv_bin=1.0 / (B * IN), out_n=OUT),
        in_specs=[vm, vm, vm, vm, vm, vm,
                  pl.BlockSpec(memory_space=pltpu.MemorySpace.HBM)],
        out_specs=[vm, vm, vm,
                   pl.BlockSpec(memory_space=pltpu.MemorySpace.HBM)],
        out_shape=[
            jax.ShapeDtypeStruct((1, OUT), f32),
            jax.ShapeDtypeStruct((1, OUT), f32),
            jax.ShapeDtypeStruct((1, OUT), ages.dtype),
            jax.ShapeDtypeStruct((OUT, IN), f32),
        ],
        scratch_shapes=[
            pltpu.VMEM((1, IN), f32),
            pltpu.SemaphoreType.DMA((_KMAX,)),
        ],
        input_output_aliases={6: 3},
    )(cas, outg_part, incoming.reshape(1, OUT), utilities.reshape(1, OUT),
      ages.reshape(1, OUT), bias.reshape(1, OUT), w_copy)

    return (weight_new, bias_new.reshape(OUT), u_new.reshape(OUT),
            rm_new2.reshape(IN), ages_new.reshape(OUT))


# speedup vs baseline: 14.8991x; 14.8991x over previous
import jax
import jax.numpy as jnp
from jax.experimental import pallas as pl
from jax.experimental.pallas import tpu as pltpu
from jax.experimental.pallas import tpu_sc as plsc

_DECAY = 0.9
_OMD = 1.0 - _DECAY
_MATURITY = 500
_REINIT_DIV = 100
_KMAX = 64
_I32MAX = 2**31 - 1


def _sc_copy_kernel(w_hbm, i_hbm, o_hbm, *, rows, blk):
    def body(x_vmem):
        step = pl.program_id(0)
        pltpu.sync_copy(x_vmem, o_hbm.at[pl.ds(step * blk, blk)])

    pltpu.emit_pipeline(
        body,
        grid=(rows // blk,),
        in_specs=[pl.BlockSpec((blk, w_hbm.shape[1]), lambda i: (i, 0))],
        out_specs=[],
        core_axis_name=("c", "s"),
        dimension_semantics=(pltpu.PARALLEL,),
    )(w_hbm)


def _stream_kernel(w_ref, f_ref, rm_ref, outg_ref, inc_ref,
                   rmnew_ref, cas_ref, *, inv_b, nwb):
    s = pl.program_id(0)

    @pl.when(s < nwb)
    def _weight_phase():
        aw = jnp.abs(w_ref[...])
        outg_ref[...] = jnp.sum(aw, axis=0)[None, None, :]
        inc_ref[...] = jnp.sum(aw, axis=1, keepdims=True)

    @pl.when(s >= nwb)
    def _features_phase():
        f = f_ref[...]
        colsum = jnp.sum(f, axis=0, keepdims=True)
        rm_new = _DECAY * rm_ref[...] + _OMD * (colsum * inv_b)
        rmnew_ref[...] = rm_new
        cas_ref[...] = jnp.sum(jnp.abs(f - rm_new), axis=0, keepdims=True)


def _excl_prefix_sum_row(x):
    n = x.shape[1]
    s = x
    sh = 1
    while sh < n:
        shifted = jnp.concatenate(
            [jnp.zeros((1, sh), x.dtype), s[:, :n - sh]], axis=1)
        s = s + shifted
        sh *= 2
    return s - x


def _update_select_kernel(cas_ref, outg_ref, incr_ref, u_ref, ages_ref,
                          bias_ref, w_ref, unew_ref, bnew_ref, anew_ref,
                          wout_ref, zeros_ref, sems, *, inv_bin, out_n):
    i32 = jnp.int32
    outgoing = jnp.sum(outg_ref[...], axis=0)
    c = jnp.sum(outgoing * cas_ref[...]) * inv_bin
    u_new = _DECAY * u_ref[...] + _OMD * (c / (incr_ref[...] + 1e-8))
    unew_ref[...] = u_new

    ages = ages_ref[...]
    mature = ages > _MATURITY
    num_mature = jnp.sum(mature.astype(i32))
    num_reinit = num_mature // _REINIT_DIV
    r = jnp.maximum(num_reinit, 1)

    key = jnp.where(mature, jax.lax.bitcast_convert_type(u_new, i32),
                    i32(_I32MAX))

    def bisect(_, lohi):
        lo, hi = lohi
        mid = (lo + hi) // 2
        ge = jnp.sum((key <= mid).astype(i32)) >= r
        return jnp.where(ge, lo, mid), jnp.where(ge, mid, hi)

    _, v = jax.lax.fori_loop(0, 31, bisect, (i32(-1), i32(_I32MAX)))

    c_lt = jnp.sum((key < v).astype(i32))
    tie = key == v
    tie_rank = _excl_prefix_sum_row(tie.astype(i32))
    sel = ((key < v) | (tie & (tie_rank < (r - c_lt)))) & (num_reinit > 0)

    bnew_ref[...] = jnp.where(sel, 0.0, bias_ref[...])
    anew_ref[...] = jnp.where(sel, 0, ages) + 1

    slot = _excl_prefix_sum_row(sel.astype(i32))
    i_row = jax.lax.broadcasted_iota(i32, (1, out_n), 1)
    zeros_ref[...] = jnp.zeros_like(zeros_ref)
    idxs = []
    for k in range(_KMAX):
        hit = sel & (slot == k)
        idx_k = jnp.sum(jnp.where(hit, i_row + 1, 0)) - 1
        idxs.append(idx_k)

        @pl.when(idx_k >= 0)
        def _():
            pltpu.make_async_copy(
                zeros_ref, wout_ref.at[pl.ds(idx_k, 1), :], sems.at[k],
            ).start()
    for k in range(_KMAX):
        @pl.when(idxs[k] >= 0)
        def _():
            pltpu.make_async_copy(
                zeros_ref, wout_ref.at[pl.ds(idxs[k], 1), :], sems.at[k],
            ).wait()


def kernel(features, weight, bias, utilities, running_mean, ages):
    B, IN = features.shape
    OUT = weight.shape[0]
    CB = 512
    RB = 256
    NB = OUT // RB
    f32 = jnp.float32

    rm2 = running_mean.reshape(1, IN)
    NWB = NB
    NFB = IN // CB

    sc_kernel = pl.kernel(
        lambda w, i, o: _sc_copy_kernel(w, i, o, rows=OUT, blk=8),
        out_type=jax.ShapeDtypeStruct((OUT, IN), f32),
        mesh=plsc.VectorSubcoreMesh(core_axis_name="c", subcore_axis_name="s"),
    )
    row_ids = jax.lax.iota(jnp.int32, OUT).reshape(1, OUT)
    w_copy = sc_kernel(weight, row_ids)

    wlast = NWB - 1
    outg_part, incoming, rm_new2, cas = pl.pallas_call(
        lambda w, f, r, o1, o2, o4, o5: _stream_kernel(
            w, f, r, o1, o2, o4, o5, inv_b=1.0 / B, nwb=NWB),
        grid=(NWB + NFB,),
        in_specs=[
            pl.BlockSpec((RB, IN), lambda s: (jnp.minimum(s, wlast), 0)),
            pl.BlockSpec((B, CB), lambda s: (0, jnp.maximum(s - NWB, 0))),
            pl.BlockSpec((1, CB), lambda s: (0, jnp.maximum(s - NWB, 0))),
        ],
        out_specs=[
            pl.BlockSpec((1, 1, IN), lambda s: (jnp.minimum(s, wlast), 0, 0)),
            pl.BlockSpec((RB, 1), lambda s: (jnp.minimum(s, wlast), 0)),
            pl.BlockSpec((1, CB), lambda s: (0, jnp.maximum(s - NWB, 0))),
            pl.BlockSpec((1, CB), lambda s: (0, jnp.maximum(s - NWB, 0))),
        ],
        out_shape=[
            jax.ShapeDtypeStruct((NB, 1, IN), f32),
            jax.ShapeDtypeStruct((OUT, 1), f32),
            jax.ShapeDtypeStruct((1, IN), f32),
            jax.ShapeDtypeStruct((1, IN), f32),
        ],
    )(weight, features, rm2)

    vm = pl.BlockSpec(memory_space=pltpu.MemorySpace.VMEM)
    u_new, bias_new, ages_new, weight_new = pl.pallas_call(
        lambda ca, og, ic, u, ag, b, w, un, bn, an, wo, zr, sm:
            _update_select_kernel(
                ca, og, ic, u, ag, b, w, un, bn, an, wo, zr, sm,
                inv_bin=1.0 / (B * IN), out_n=OUT),
        in_specs=[vm, vm, vm, vm, vm, vm,
                  pl.BlockSpec(memory_space=pltpu.MemorySpace.HBM)],
        out_specs=[vm, vm, vm,
                   pl.BlockSpec(memory_space=pltpu.MemorySpace.HBM)],
        out_shape=[
            jax.ShapeDtypeStruct((1, OUT), f32),
            jax.ShapeDtypeStruct((1, OUT), f32),
            jax.ShapeDtypeStruct((1, OUT), ages.dtype),
            jax.ShapeDtypeStruct((OUT, IN), f32),
        ],
        scratch_shapes=[
            pltpu.VMEM((1, IN), f32),
            pltpu.SemaphoreType.DMA((_KMAX,)),
        ],
        input_output_aliases={6: 3},
    )(cas, outg_part, incoming.reshape(1, OUT), utilities.reshape(1, OUT),
      ages.reshape(1, OUT), bias.reshape(1, OUT), w_copy)

    return (weight_new, bias_new.reshape(OUT), u_new.reshape(OUT),
            rm_new2.reshape(IN), ages_new.reshape(OUT))


# speedup vs baseline: 20.3230x; 1.3640x over previous
import jax
import jax.numpy as jnp
from jax.experimental import pallas as pl
from jax.experimental.pallas import tpu as pltpu

_DECAY = 0.9
_OMD = 1.0 - _DECAY
_MATURITY = 500
_REINIT_DIV = 100
_KMAX = 64
_I32MAX = 2**31 - 1


def _stream_kernel(w_ref, f_ref, rm_ref, outg_ref, inc_ref, wcopy_ref,
                   rmnew_ref, cas_ref, *, inv_b, nwb):
    s = pl.program_id(0)

    @pl.when(s < nwb)
    def _weight_phase():
        w = w_ref[...]
        wcopy_ref[...] = w
        aw = jnp.abs(w)
        outg_ref[...] = jnp.sum(aw, axis=0)[None, None, :]
        inc_ref[...] = jnp.sum(aw, axis=1, keepdims=True)

    @pl.when(s >= nwb)
    def _features_phase():
        f = f_ref[...]
        colsum = jnp.sum(f, axis=0, keepdims=True)
        rm_new = _DECAY * rm_ref[...] + _OMD * (colsum * inv_b)
        rmnew_ref[...] = rm_new
        cas_ref[...] = jnp.sum(jnp.abs(f - rm_new), axis=0, keepdims=True)


def _excl_prefix_sum_row(x):
    n = x.shape[1]
    s = x
    sh = 1
    while sh < n:
        shifted = jnp.concatenate(
            [jnp.zeros((1, sh), x.dtype), s[:, :n - sh]], axis=1)
        s = s + shifted
        sh *= 2
    return s - x


def _update_select_kernel(cas_ref, outg_ref, incr_ref, u_ref, ages_ref,
                          bias_ref, w_ref, unew_ref, bnew_ref, anew_ref,
                          wout_ref, zeros_ref, sems, *, inv_bin, out_n):
    i32 = jnp.int32
    outgoing = jnp.sum(outg_ref[...], axis=0)
    c = jnp.sum(outgoing * cas_ref[...]) * inv_bin
    u_new = _DECAY * u_ref[...] + _OMD * (c / (incr_ref[...] + 1e-8))
    unew_ref[...] = u_new

    ages = ages_ref[...]
    mature = ages > _MATURITY
    num_mature = jnp.sum(mature.astype(i32))
    num_reinit = num_mature // _REINIT_DIV
    r = jnp.maximum(num_reinit, 1)

    key = jnp.where(mature, jax.lax.bitcast_convert_type(u_new, i32),
                    i32(_I32MAX))

    def bisect(_, lohi):
        lo, hi = lohi
        mid = (lo + hi) // 2
        ge = jnp.sum((key <= mid).astype(i32)) >= r
        return jnp.where(ge, lo, mid), jnp.where(ge, mid, hi)

    _, v = jax.lax.fori_loop(0, 31, bisect, (i32(-1), i32(_I32MAX)))

    c_lt = jnp.sum((key < v).astype(i32))
    tie = key == v
    tie_rank = _excl_prefix_sum_row(tie.astype(i32))
    sel = ((key < v) | (tie & (tie_rank < (r - c_lt)))) & (num_reinit > 0)

    bnew_ref[...] = jnp.where(sel, 0.0, bias_ref[...])
    anew_ref[...] = jnp.where(sel, 0, ages) + 1

    slot = _excl_prefix_sum_row(sel.astype(i32))
    i_row = jax.lax.broadcasted_iota(i32, (1, out_n), 1)
    zeros_ref[...] = jnp.zeros_like(zeros_ref)
    idxs = []
    for k in range(_KMAX):
        hit = sel & (slot == k)
        idx_k = jnp.sum(jnp.where(hit, i_row + 1, 0)) - 1
        idxs.append(idx_k)

        @pl.when(idx_k >= 0)
        def _():
            pltpu.make_async_copy(
                zeros_ref, wout_ref.at[pl.ds(idx_k, 1), :], sems.at[k],
            ).start()
    for k in range(_KMAX):
        @pl.when(idxs[k] >= 0)
        def _():
            pltpu.make_async_copy(
                zeros_ref, wout_ref.at[pl.ds(idxs[k], 1), :], sems.at[k],
            ).wait()


def kernel(features, weight, bias, utilities, running_mean, ages):
    B, IN = features.shape
    OUT = weight.shape[0]
    CB = 512
    RB = 256
    NB = OUT // RB
    f32 = jnp.float32

    rm2 = running_mean.reshape(1, IN)
    NWB = NB
    NFB = IN // CB

    wlast = NWB - 1
    outg_part, incoming, w_copy, rm_new2, cas = pl.pallas_call(
        lambda w, f, r, o1, o2, o3, o4, o5: _stream_kernel(
            w, f, r, o1, o2, o3, o4, o5, inv_b=1.0 / B, nwb=NWB),
        grid=(NWB + NFB,),
        in_specs=[
            pl.BlockSpec((RB, IN), lambda s: (jnp.minimum(s, wlast), 0)),
            pl.BlockSpec((B, CB), lambda s: (0, jnp.maximum(s - NWB, 0))),
            pl.BlockSpec((1, CB), lambda s: (0, jnp.maximum(s - NWB, 0))),
        ],
        out_specs=[
            pl.BlockSpec((1, 1, IN), lambda s: (jnp.minimum(s, wlast), 0, 0)),
            pl.BlockSpec((RB, 1), lambda s: (jnp.minimum(s, wlast), 0)),
            pl.BlockSpec((RB, IN), lambda s: (jnp.minimum(s, wlast), 0)),
            pl.BlockSpec((1, CB), lambda s: (0, jnp.maximum(s - NWB, 0))),
            pl.BlockSpec((1, CB), lambda s: (0, jnp.maximum(s - NWB, 0))),
        ],
        out_shape=[
            jax.ShapeDtypeStruct((NB, 1, IN), f32),
            jax.ShapeDtypeStruct((OUT, 1), f32),
            jax.ShapeDtypeStruct((OUT, IN), f32),
            jax.ShapeDtypeStruct((1, IN), f32),
            jax.ShapeDtypeStruct((1, IN), f32),
        ],
    )(weight, features, rm2)

    vm = pl.BlockSpec(memory_space=pltpu.MemorySpace.VMEM)
    u_new, bias_new, ages_new, weight_new = pl.pallas_call(
        lambda ca, og, ic, u, ag, b, w, un, bn, an, wo, zr, sm:
            _update_select_kernel(
                ca, og, ic, u, ag, b, w, un, bn, an, wo, zr, sm,
                inv_bin=1.0 / (B * IN), out_n=OUT),
        in_specs=[vm, vm, vm, vm, vm, vm,
                  pl.BlockSpec(memory_space=pltpu.MemorySpace.HBM)],
        out_specs=[vm, vm, vm,
                   pl.BlockSpec(memory_space=pltpu.MemorySpace.HBM)],
        out_shape=[
            jax.ShapeDtypeStruct((1, OUT), f32),
            jax.ShapeDtypeStruct((1, OUT), f32),
            jax.ShapeDtypeStruct((1, OUT), ages.dtype),
            jax.ShapeDtypeStruct((OUT, IN), f32),
        ],
        scratch_shapes=[
            pltpu.VMEM((1, IN), f32),
            pltpu.SemaphoreType.DMA((_KMAX,)),
        ],
        input_output_aliases={6: 3},
    )(cas, outg_part, incoming.reshape(1, OUT), utilities.reshape(1, OUT),
      ages.reshape(1, OUT), bias.reshape(1, OUT), w_copy)

    return (weight_new, bias_new.reshape(OUT), u_new.reshape(OUT),
            rm_new2.reshape(IN), ages_new.reshape(OUT))


# speedup vs baseline: 20.5232x; 1.0098x over previous
import jax
import jax.numpy as jnp
from jax.experimental import pallas as pl
from jax.experimental.pallas import tpu as pltpu

_DECAY = 0.9
_OMD = 1.0 - _DECAY
_MATURITY = 500
_REINIT_DIV = 100
_KMAX = 64
_I32MAX = 2**31 - 1


def _feat_kernel(f_ref, rm_ref, rmnew_ref, cas_ref, *, inv_b):
    f = f_ref[...]
    colsum = jnp.sum(f, axis=0, keepdims=True)
    rm_new = _DECAY * rm_ref[...] + _OMD * (colsum * inv_b)
    rmnew_ref[...] = rm_new
    cas_ref[...] = jnp.sum(jnp.abs(f - rm_new), axis=0, keepdims=True)


def _wsum_copy_kernel(w_ref, outg_ref, inc_ref, wcopy_ref):
    w = w_ref[...]
    wcopy_ref[...] = w
    aw = jnp.abs(w)
    outg_ref[...] = jnp.sum(aw, axis=0)[None, None, :]
    inc_ref[...] = jnp.sum(aw, axis=1, keepdims=True)


def _excl_prefix_sum_row(x):
    n = x.shape[1]
    s = x
    sh = 1
    while sh < n:
        shifted = jnp.concatenate(
            [jnp.zeros((1, sh), x.dtype), s[:, :n - sh]], axis=1)
        s = s + shifted
        sh *= 2
    return s - x


def _update_select_kernel(cas_ref, outg_ref, incr_ref, u_ref, ages_ref,
                          bias_ref, w_ref, unew_ref, bnew_ref, anew_ref,
                          wout_ref, zeros_ref, sems, *, inv_bin, out_n):
    i32 = jnp.int32
    outgoing = jnp.sum(outg_ref[...], axis=0)
    c = jnp.sum(outgoing * cas_ref[...]) * inv_bin
    u_new = _DECAY * u_ref[...] + _OMD * (c / (incr_ref[...] + 1e-8))
    unew_ref[...] = u_new

    ages = ages_ref[...]
    mature = ages > _MATURITY
    num_mature = jnp.sum(mature.astype(i32))
    num_reinit = num_mature // _REINIT_DIV
    r = jnp.maximum(num_reinit, 1)

    key = jnp.where(mature, jax.lax.bitcast_convert_type(u_new, i32),
                    i32(_I32MAX))

    def bisect(_, lohi):
        lo, hi = lohi
        mid = (lo + hi) // 2
        ge = jnp.sum((key <= mid).astype(i32)) >= r
        return jnp.where(ge, lo, mid), jnp.where(ge, mid, hi)

    _, v = jax.lax.fori_loop(0, 31, bisect, (i32(-1), i32(_I32MAX)))

    c_lt = jnp.sum((key < v).astype(i32))
    tie = key == v
    tie_rank = _excl_prefix_sum_row(tie.astype(i32))
    sel = ((key < v) | (tie & (tie_rank < (r - c_lt)))) & (num_reinit > 0)

    bnew_ref[...] = jnp.where(sel, 0.0, bias_ref[...])
    anew_ref[...] = jnp.where(sel, 0, ages) + 1

    slot = _excl_prefix_sum_row(sel.astype(i32))
    i_row = jax.lax.broadcasted_iota(i32, (1, out_n), 1)
    zeros_ref[...] = jnp.zeros_like(zeros_ref)
    idxs = []
    for k in range(_KMAX):
        hit = sel & (slot == k)
        idx_k = jnp.sum(jnp.where(hit, i_row + 1, 0)) - 1
        idxs.append(idx_k)

        @pl.when(idx_k >= 0)
        def _():
            pltpu.make_async_copy(
                zeros_ref, wout_ref.at[pl.ds(idx_k, 1), :], sems.at[k],
            ).start()
    for k in range(_KMAX):
        @pl.when(idxs[k] >= 0)
        def _():
            pltpu.make_async_copy(
                zeros_ref, wout_ref.at[pl.ds(idxs[k], 1), :], sems.at[k],
            ).wait()


def kernel(features, weight, bias, utilities, running_mean, ages):
    B, IN = features.shape
    OUT = weight.shape[0]
    CB = 512
    RB = 512
    NB = OUT // RB
    f32 = jnp.float32

    rm2 = running_mean.reshape(1, IN)

    rm_new2, cas = pl.pallas_call(
        lambda f, r, o1, o2: _feat_kernel(f, r, o1, o2, inv_b=1.0 / B),
        grid=(IN // CB,),
        in_specs=[
            pl.BlockSpec((B, CB), lambda j: (0, j)),
            pl.BlockSpec((1, CB), lambda j: (0, j)),
        ],
        out_specs=[
            pl.BlockSpec((1, CB), lambda j: (0, j)),
            pl.BlockSpec((1, CB), lambda j: (0, j)),
        ],
        out_shape=[
            jax.ShapeDtypeStruct((1, IN), f32),
            jax.ShapeDtypeStruct((1, IN), f32),
        ],
        compiler_params=pltpu.CompilerParams(
            dimension_semantics=("parallel",)),
    )(features, rm2)

    outg_part, incoming, w_copy = pl.pallas_call(
        _wsum_copy_kernel,
        grid=(NB,),
        in_specs=[pl.BlockSpec((RB, IN), lambda i: (i, 0))],
        out_specs=[
            pl.BlockSpec((1, 1, IN), lambda i: (i, 0, 0)),
            pl.BlockSpec((RB, 1), lambda i: (i, 0)),
            pl.BlockSpec((RB, IN), lambda i: (i, 0)),
        ],
        out_shape=[
            jax.ShapeDtypeStruct((NB, 1, IN), f32),
            jax.ShapeDtypeStruct((OUT, 1), f32),
            jax.ShapeDtypeStruct((OUT, IN), f32),
        ],
        compiler_params=pltpu.CompilerParams(
            dimension_semantics=("parallel",)),
    )(weight)

    vm = pl.BlockSpec(memory_space=pltpu.MemorySpace.VMEM)
    u_new, bias_new, ages_new, weight_new = pl.pallas_call(
        lambda ca, og, ic, u, ag, b, w, un, bn, an, wo, zr, sm:
            _update_select_kernel(
                ca, og, ic, u, ag, b, w, un, bn, an, wo, zr, sm,
                inv_bin=1.0 / (B * IN), out_n=OUT),
        in_specs=[vm, vm, vm, vm, vm, vm,
                  pl.BlockSpec(memory_space=pltpu.MemorySpace.HBM)],
        out_specs=[vm, vm, vm,
                   pl.BlockSpec(memory_space=pltpu.MemorySpace.HBM)],
        out_shape=[
            jax.ShapeDtypeStruct((1, OUT), f32),
            jax.ShapeDtypeStruct((1, OUT), f32),
            jax.ShapeDtypeStruct((1, OUT), ages.dtype),
            jax.ShapeDtypeStruct((OUT, IN), f32),
        ],
        scratch_shapes=[
            pltpu.VMEM((1, IN), f32),
            pltpu.SemaphoreType.DMA((_KMAX,)),
        ],
        input_output_aliases={6: 3},
    )(cas, outg_part, incoming.reshape(1, OUT), utilities.reshape(1, OUT),
      ages.reshape(1, OUT), bias.reshape(1, OUT), w_copy)

    return (weight_new, bias_new.reshape(OUT), u_new.reshape(OUT),
            rm_new2.reshape(IN), ages_new.reshape(OUT))
